# baseline (speedup 1.0000x reference)
import functools

import jax
import jax.numpy as jnp
from jax import lax
from jax.experimental import pallas as pl
from jax.experimental.pallas import tpu as pltpu
from jax.experimental.pallas import tpu_sc as plsc

B, S, D = 16, 2048, 128
NC, NS = 2, 16
NW = NC * NS
P = S // NW
RV = D // 16
NBUF = 6
AHEAD = 3


def _emb_body(x_hbm, tok_hbm, pos_hbm, out_hbm, idx_v, pos_v, *rest):
    bufs, gsems, wsems = rest[:NBUF], rest[NBUF:2 * NBUF], rest[2 * NBUF:]
    wid = lax.axis_index("s") * NC + lax.axis_index("c")
    p0 = wid * P
    ih = [pltpu.async_copy(x_hbm.at[b, pl.ds(p0, P)], idx_v.at[b], wsems[0])
          for b in range(B)]
    pltpu.sync_copy(pos_hbm.at[pl.ds(p0, P)], pos_v)
    for h in ih:
        h.wait()

    gets, puts = {}, {}

    def start_gather(c):
        gets[c] = pltpu.async_copy(tok_hbm.at[idx_v.at[c]], bufs[c % NBUF],
                                   gsems[c % NBUF])

    for c in range(min(AHEAD, B)):
        start_gather(c)
    for b in range(B):
        nb = b + AHEAD
        if nb < B:
            if nb >= NBUF:
                puts[nb - NBUF].wait()
            start_gather(nb)
        gets[b].wait()
        cur = bufs[b % NBUF]

        def add_row(i, carry, cur=cur):
            for j in range(RV):
                sl = pl.ds(j * 16, 16)
                plsc.addupdate(cur.at[i, sl], pos_v[i, sl])
            return carry

        lax.fori_loop(0, P, add_row, 0)
        puts[b] = pltpu.async_copy(cur, out_hbm.at[b, pl.ds(p0, P)],
                                   wsems[b % NBUF])
    for b in range(max(0, B - NBUF), B):
        puts[b].wait()


_emb_kernel = functools.partial(
    pl.kernel,
    mesh=plsc.VectorSubcoreMesh(core_axis_name="c", subcore_axis_name="s"),
    out_type=jax.ShapeDtypeStruct((B, S, D), jnp.float32),
    scratch_types=(
        [pltpu.VMEM((B, P), jnp.int32), pltpu.VMEM((P, D), jnp.float32)]
        + [pltpu.VMEM((P, D), jnp.float32) for _ in range(NBUF)]
        + [pltpu.SemaphoreType.DMA for _ in range(2 * NBUF)]
    ),
)(_emb_body)


def kernel(x, token_table, pos_table):
    return _emb_kernel(x.astype(jnp.int32), token_table, pos_table)
